# baseline (speedup 1.0000x reference)
import functools

import jax
import jax.numpy as jnp
from jax import lax
from jax.experimental import pallas as pl
from jax.experimental.pallas import tpu as pltpu
from jax.experimental.pallas import tpu_sc as plsc

N = 10000
D = 128
NPAD = 10240
NC, NS, L = 2, 16, 16
NW = NC * NS
K = 128
KA = 64
NBUF = 3
ROWS_PER_TILE = NPAD // NS


def _sc_mesh():
    return plsc.VectorSubcoreMesh(core_axis_name="c", subcore_axis_name="s")


def _make_deg_kernel(nb):
    @functools.partial(
        pl.kernel,
        out_type=jax.ShapeDtypeStruct((NC, 2, NPAD), jnp.float32),
        mesh=_sc_mesh(),
        scratch_types=[
            pltpu.VMEM((nb, 2, K), jnp.int32),
            pltpu.VMEM((nb, K), jnp.float32),
            pltpu.VMEM((ROWS_PER_TILE,), jnp.float32),
            pltpu.VMEM_SHARED((NPAD,), jnp.float32),
            pltpu.VMEM_SHARED((NPAD,), jnp.float32),
        ],
    )
    def k(sd_hbm, w_hbm, out_hbm, sd_v, w_v, zero_v, dego_sp, degi_sp):
        cid = lax.axis_index("c")
        sid = lax.axis_index("s")
        wid = cid * NS + sid

        zero = jnp.zeros((L,), jnp.float32)

        def zloop(i, _):
            zero_v[pl.ds(i * L, L)] = zero
            return 0

        lax.fori_loop(0, ROWS_PER_TILE // L, zloop, 0)
        pltpu.sync_copy(zero_v,
                        dego_sp.at[pl.ds(sid * ROWS_PER_TILE, ROWS_PER_TILE)])
        pltpu.sync_copy(zero_v,
                        degi_sp.at[pl.ds(sid * ROWS_PER_TILE, ROWS_PER_TILE)])
        pltpu.sync_copy(sd_hbm.at[wid], sd_v)
        pltpu.sync_copy(w_hbm.at[wid], w_v)
        plsc.subcore_barrier()

        def body(b, _):
            pltpu.sync_copy(w_v.at[b], dego_sp.at[sd_v.at[b, 0]], add=True)
            pltpu.sync_copy(w_v.at[b], degi_sp.at[sd_v.at[b, 1]], add=True)
            return 0

        lax.fori_loop(0, nb, body, 0)
        plsc.subcore_barrier()

        @pl.when(sid == 0)
        def _():
            pltpu.sync_copy(dego_sp, out_hbm.at[cid, 0])
            pltpu.sync_copy(degi_sp, out_hbm.at[cid, 1])

    return k


def _make_agg_kernel(nb):
    assert nb % NBUF == 0 and nb >= 2 * NBUF
    m = nb // NBUF

    @functools.partial(
        pl.kernel,
        out_type=jax.ShapeDtypeStruct((NC, NPAD, D), jnp.float32),
        mesh=_sc_mesh(),
        scratch_types=[
            pltpu.VMEM((nb, KA), jnp.int32),
            pltpu.VMEM((KA, D), jnp.float32),
            pltpu.VMEM((KA, D), jnp.float32),
            pltpu.VMEM((KA, D), jnp.float32),
            pltpu.VMEM((NBUF, KA), jnp.int32),
            pltpu.VMEM((NBUF, KA), jnp.float32),
            pltpu.VMEM_SHARED((NPAD, D), jnp.float32),
            pltpu.SemaphoreType.DMA,
            pltpu.SemaphoreType.DMA,
            pltpu.SemaphoreType.DMA,
            pltpu.SemaphoreType.DMA,
            pltpu.SemaphoreType.DMA,
            pltpu.SemaphoreType.DMA,
        ],
    )
    def k(h_hbm, src_hbm, dst_hbm, w_hbm, out_hbm, src_v, a0, a1, a2,
          dring, wring, agg_sp, sg0, sg1, sg2, se0, se1, se2):
        cid = lax.axis_index("c")
        sid = lax.axis_index("s")
        wid = cid * NS + sid
        abuf = (a0, a1, a2)
        gsem = (sg0, sg1, sg2)
        esem = (se0, se1, se2)

        def fetch_e(blk, slot):
            pltpu.async_copy(dst_hbm.at[wid, blk], dring.at[slot],
                             esem[slot])
            pltpu.async_copy(w_hbm.at[wid, blk], wring.at[slot], esem[slot])

        def wait_e(slot):
            pltpu.make_async_copy(dst_hbm.at[wid, 0], dring.at[slot],
                                  esem[slot]).wait()
            pltpu.make_async_copy(w_hbm.at[wid, 0], wring.at[slot],
                                  esem[slot]).wait()

        def gather(blk, u):
            pltpu.async_copy(h_hbm.at[src_v.at[blk]], abuf[u], gsem[u])

        def wait_g(blk, u):
            pltpu.make_async_copy(h_hbm.at[src_v.at[blk]], abuf[u],
                                  gsem[u]).wait()

        zero = jnp.zeros((L,), jnp.float32)

        def zloop(r, _):
            for f in range(D // L):
                a0[r, pl.ds(f * L, L)] = zero
            return 0

        lax.fori_loop(0, KA, zloop, 0)
        for i in range(ROWS_PER_TILE // KA):
            pltpu.sync_copy(
                a0, agg_sp.at[pl.ds(sid * ROWS_PER_TILE + i * KA, KA)])
        pltpu.sync_copy(src_hbm.at[wid], src_v)
        plsc.subcore_barrier()

        gdn = lax.GatherDimensionNumbers(
            offset_dims=(), collapsed_slice_dims=(0,), start_index_map=(0,))

        for jj in range(NBUF):
            fetch_e(jj, jj)
            gather(jj, jj)

        def body(g, _):
            for u in range(NBUF):
                b = g * NBUF + u
                av = abuf[u]
                wait_g(b, u)
                wait_e(u)

                def scale(gr, _):
                    w16 = wring[u, pl.ds(gr * L, L)]
                    for j in range(L):
                        wb = lax.gather(
                            w16, jnp.full((L, 1), j, jnp.int32), gdn,
                            slice_sizes=(1,),
                            mode=lax.GatherScatterMode.PROMISE_IN_BOUNDS)
                        r = gr * L + j
                        for f in range(D // L):
                            av[r, pl.ds(f * L, L)] = (
                                av[r, pl.ds(f * L, L)] * wb)
                    return 0

                lax.fori_loop(0, KA // L, scale, 0)
                pltpu.sync_copy(av, agg_sp.at[dring.at[u]], add=True)

                @pl.when(g < m - 1)
                def _():
                    fetch_e(b + NBUF, u)
                    gather(b + NBUF, u)
            return 0

        lax.fori_loop(0, m, body, 0)
        plsc.subcore_barrier()
        pltpu.sync_copy(
            agg_sp.at[pl.ds(sid * ROWS_PER_TILE, ROWS_PER_TILE)],
            out_hbm.at[cid, pl.ds(sid * ROWS_PER_TILE, ROWS_PER_TILE)])

    return k


BLK = 2000


def _dinv(ref):
    return lax.rsqrt(jnp.maximum(ref[:, 0:1] + ref[:, 1:2], 1e-12))


def _tc_first_body(x_ref, w_ref, go_ref, o_ref):
    h = jnp.dot(x_ref[...], w_ref[...], preferred_element_type=jnp.float32)
    o_ref[...] = h * _dinv(go_ref)


def _tc_first(x, w, dego):
    return pl.pallas_call(
        _tc_first_body,
        out_shape=jax.ShapeDtypeStruct((N, D), jnp.float32),
        grid=(N // BLK,),
        in_specs=[
            pl.BlockSpec((BLK, D), lambda i: (i, 0)),
            pl.BlockSpec((D, D), lambda i: (0, 0)),
            pl.BlockSpec((BLK, 2), lambda i: (i, 0)),
        ],
        out_specs=pl.BlockSpec((BLK, D), lambda i: (i, 0)),
    )(x, w, dego)


def _tc_mid_body(p_ref, gi_ref, go_ref, b_ref, w_ref, o_ref):
    agg = (p_ref[0] + p_ref[1]) * _dinv(gi_ref)
    x = jnp.maximum(agg + b_ref[...], 0.0)
    h = jnp.dot(x, w_ref[...], preferred_element_type=jnp.float32)
    o_ref[...] = h * _dinv(go_ref)


def _tc_mid(parts, degi, dego, b, w):
    return pl.pallas_call(
        _tc_mid_body,
        out_shape=jax.ShapeDtypeStruct((N, D), jnp.float32),
        grid=(N // BLK,),
        in_specs=[
            pl.BlockSpec((NC, BLK, D), lambda i: (0, i, 0)),
            pl.BlockSpec((BLK, 2), lambda i: (i, 0)),
            pl.BlockSpec((BLK, 2), lambda i: (i, 0)),
            pl.BlockSpec((1, D), lambda i: (0, 0)),
            pl.BlockSpec((D, D), lambda i: (0, 0)),
        ],
        out_specs=pl.BlockSpec((BLK, D), lambda i: (i, 0)),
    )(parts, degi, dego, b, w)


def _tc_final_body(p_ref, gi_ref, b_ref, o_ref):
    agg = (p_ref[0] + p_ref[1]) * _dinv(gi_ref)
    x = jnp.maximum(agg + b_ref[...], 0.0)
    m = jnp.max(x, axis=1, keepdims=True)
    e = jnp.exp(x - m)
    o_ref[...] = e / jnp.sum(e, axis=1, keepdims=True)


def _tc_final(parts, degi, b):
    return pl.pallas_call(
        _tc_final_body,
        out_shape=jax.ShapeDtypeStruct((N, D), jnp.float32),
        grid=(N // BLK,),
        in_specs=[
            pl.BlockSpec((NC, BLK, D), lambda i: (0, i, 0)),
            pl.BlockSpec((BLK, 2), lambda i: (i, 0)),
            pl.BlockSpec((1, D), lambda i: (0, 0)),
        ],
        out_specs=pl.BlockSpec((BLK, D), lambda i: (i, 0)),
    )(parts, degi, b)


def kernel(in_feat, edge_index, edge_weight, W0, b0, W1, b1, W2, b2):
    src = edge_index[0]
    dst = edge_index[1]
    e = edge_weight.shape[0]
    nb = -(-e // (NW * K))
    nb += (-nb) % NBUF
    epad = NW * nb * K - e

    def pad(x):
        return jnp.pad(x, (0, epad)).reshape(NW, nb, 1, K)

    sdpad = jnp.concatenate([pad(src), pad(dst)], axis=2)
    wpad = jnp.pad(edge_weight, (0, epad)).reshape(NW, nb, K)
    deg = _make_deg_kernel(nb)(sdpad, wpad)

    nba = nb * K // KA
    srca = sdpad[:, :, 0].reshape(NW, nba, KA)
    dsta = sdpad[:, :, 1].reshape(NW, nba, KA)
    wa = wpad.reshape(NW, nba, KA)
    dego = jnp.stack([deg[0, 0], deg[1, 0]], axis=1)
    degi = jnp.stack([deg[0, 1], deg[1, 1]], axis=1)
    b0r = b0.reshape(1, D)
    b1r = b1.reshape(1, D)
    b2r = b2.reshape(1, D)

    agg_k = _make_agg_kernel(nba)
    h = _tc_first(in_feat, W0, dego)
    p = agg_k(h, srca, dsta, wa)
    h = _tc_mid(p, degi, dego, b0r, W1)
    p = agg_k(h, srca, dsta, wa)
    h = _tc_mid(p, degi, dego, b1r, W2)
    p = agg_k(h, srca, dsta, wa)
    return _tc_final(p, degi, b2r)


# speedup vs baseline: 1.3990x; 1.3990x over previous
import functools

import jax
import jax.numpy as jnp
from jax import lax
from jax.experimental import pallas as pl
from jax.experimental.pallas import tpu as pltpu
from jax.experimental.pallas import tpu_sc as plsc

N = 10000
D = 128
NPAD = 10240
NC, NS, L = 2, 16, 16
NW = NC * NS
K = 128
KA = 64
NBUF = 3
ROWS_PER_TILE = NPAD // NS


def _sc_mesh():
    return plsc.VectorSubcoreMesh(core_axis_name="c", subcore_axis_name="s")


def _make_deg_kernel(nb):
    @functools.partial(
        pl.kernel,
        out_type=jax.ShapeDtypeStruct((NC, 2, NPAD), jnp.float32),
        mesh=_sc_mesh(),
        scratch_types=[
            pltpu.VMEM((nb, 2, K), jnp.int32),
            pltpu.VMEM((nb, K), jnp.float32),
            pltpu.VMEM((ROWS_PER_TILE,), jnp.float32),
            pltpu.VMEM_SHARED((NPAD,), jnp.float32),
            pltpu.VMEM_SHARED((NPAD,), jnp.float32),
        ],
    )
    def k(sd_hbm, w_hbm, out_hbm, sd_v, w_v, zero_v, dego_sp, degi_sp):
        cid = lax.axis_index("c")
        sid = lax.axis_index("s")
        wid = cid * NS + sid

        zero = jnp.zeros((L,), jnp.float32)

        def zloop(i, _):
            zero_v[pl.ds(i * L, L)] = zero
            return 0

        lax.fori_loop(0, ROWS_PER_TILE // L, zloop, 0)
        pltpu.sync_copy(zero_v,
                        dego_sp.at[pl.ds(sid * ROWS_PER_TILE, ROWS_PER_TILE)])
        pltpu.sync_copy(zero_v,
                        degi_sp.at[pl.ds(sid * ROWS_PER_TILE, ROWS_PER_TILE)])
        pltpu.sync_copy(sd_hbm.at[wid], sd_v)
        pltpu.sync_copy(w_hbm.at[wid], w_v)
        plsc.subcore_barrier()

        def body(b, _):
            pltpu.sync_copy(w_v.at[b], dego_sp.at[sd_v.at[b, 0]], add=True)
            pltpu.sync_copy(w_v.at[b], degi_sp.at[sd_v.at[b, 1]], add=True)
            return 0

        lax.fori_loop(0, nb, body, 0)
        plsc.subcore_barrier()

        @pl.when(sid == 0)
        def _():
            pltpu.sync_copy(dego_sp, out_hbm.at[cid, 0])
            pltpu.sync_copy(degi_sp, out_hbm.at[cid, 1])

    return k


def _make_agg_kernel(nb):
    assert nb % 4 == 0 and nb >= 8
    m2 = nb // 2

    @functools.partial(
        pl.kernel,
        out_type=jax.ShapeDtypeStruct((NC, NPAD, D), jnp.float32),
        mesh=_sc_mesh(),
        scratch_types=[
            pltpu.VMEM((nb, K), jnp.int32),
            pltpu.VMEM((K, D), jnp.float32),
            pltpu.VMEM((K, D), jnp.float32),
            pltpu.VMEM((2, 2, K), jnp.int32),
            pltpu.VMEM((2, 2, K), jnp.float32),
            pltpu.VMEM_SHARED((NPAD, D), jnp.float32),
            pltpu.SemaphoreType.DMA,
            pltpu.SemaphoreType.DMA,
            pltpu.SemaphoreType.DMA,
            pltpu.SemaphoreType.DMA,
        ],
    )
    def k(h_hbm, src_hbm, dst_hbm, w_hbm, out_hbm, src_v, a0, a1,
          dslab, wslab, agg_sp, sg0, sg1, sf0, sf1):
        cid = lax.axis_index("c")
        sid = lax.axis_index("s")
        wid = cid * NS + sid
        abuf = (a0, a1)
        gsem = (sg0, sg1)
        fsem = (sf0, sf1)

        def fetch_slab(pair, slot):
            pltpu.async_copy(dst_hbm.at[wid, pair], dslab.at[slot],
                             fsem[slot])
            pltpu.async_copy(w_hbm.at[wid, pair], wslab.at[slot],
                             fsem[slot])

        def wait_slab(slot):
            pltpu.make_async_copy(dst_hbm.at[wid, 0], dslab.at[slot],
                                  fsem[slot]).wait()
            pltpu.make_async_copy(w_hbm.at[wid, 0], wslab.at[slot],
                                  fsem[slot]).wait()

        def gather(blk, u):
            pltpu.async_copy(h_hbm.at[src_v.at[blk]], abuf[u], gsem[u])

        def wait_g(blk, u):
            pltpu.make_async_copy(h_hbm.at[src_v.at[blk]], abuf[u],
                                  gsem[u]).wait()

        zero = jnp.zeros((L,), jnp.float32)

        def zloop(r, _):
            for f in range(D // L):
                a0[r, pl.ds(f * L, L)] = zero
            return 0

        lax.fori_loop(0, K, zloop, 0)
        for i in range(ROWS_PER_TILE // K):
            pltpu.sync_copy(
                a0, agg_sp.at[pl.ds(sid * ROWS_PER_TILE + i * K, K)])
        pltpu.sync_copy(src_hbm.at[wid], src_v)
        plsc.subcore_barrier()

        gdn = lax.GatherDimensionNumbers(
            offset_dims=(), collapsed_slice_dims=(0,), start_index_map=(0,))

        fetch_slab(0, 0)
        fetch_slab(1, 1)
        gather(0, 0)
        gather(1, 1)

        m4 = m2 // 2

        def body(t, _):
            for qq in range(2):
                p = t * 2 + qq
                wait_slab(qq)
                for u in range(2):
                    b = p * 2 + u
                    av = abuf[u]
                    wait_g(b, u)

                    def scale(gr, _):
                        w16 = wslab[qq, u, pl.ds(gr * L, L)]
                        for j in range(L):
                            wb = lax.gather(
                                w16, jnp.full((L, 1), j, jnp.int32), gdn,
                                slice_sizes=(1,),
                                mode=lax.GatherScatterMode.PROMISE_IN_BOUNDS)
                            r = gr * L + j
                            for f in range(D // L):
                                av[r, pl.ds(f * L, L)] = (
                                    av[r, pl.ds(f * L, L)] * wb)
                        return 0

                    lax.fori_loop(0, K // L, scale, 0)
                    pltpu.sync_copy(av, agg_sp.at[dslab.at[qq, u]],
                                    add=True)

                    if qq == 0:
                        gather(b + 2, u)
                    else:
                        @pl.when(t < m4 - 1)
                        def _():
                            gather(b + 2, u)

                @pl.when(p < m2 - 2)
                def _():
                    fetch_slab(p + 2, qq)
            return 0

        lax.fori_loop(0, m4, body, 0)
        plsc.subcore_barrier()
        pltpu.sync_copy(
            agg_sp.at[pl.ds(sid * ROWS_PER_TILE, ROWS_PER_TILE)],
            out_hbm.at[cid, pl.ds(sid * ROWS_PER_TILE, ROWS_PER_TILE)])

    return k


BLK = 2000


def _dinv(ref):
    return lax.rsqrt(jnp.maximum(ref[:, 0:1] + ref[:, 1:2], 1e-12))


def _tc_first_body(x_ref, w_ref, go_ref, o_ref):
    h = jnp.dot(x_ref[...], w_ref[...], preferred_element_type=jnp.float32)
    o_ref[...] = h * _dinv(go_ref)


def _tc_first(x, w, dego):
    return pl.pallas_call(
        _tc_first_body,
        out_shape=jax.ShapeDtypeStruct((N, D), jnp.float32),
        grid=(N // BLK,),
        in_specs=[
            pl.BlockSpec((BLK, D), lambda i: (i, 0)),
            pl.BlockSpec((D, D), lambda i: (0, 0)),
            pl.BlockSpec((BLK, 2), lambda i: (i, 0)),
        ],
        out_specs=pl.BlockSpec((BLK, D), lambda i: (i, 0)),
    )(x, w, dego)


def _tc_mid_body(p_ref, gi_ref, go_ref, b_ref, w_ref, o_ref):
    agg = (p_ref[0] + p_ref[1]) * _dinv(gi_ref)
    x = jnp.maximum(agg + b_ref[...], 0.0)
    h = jnp.dot(x, w_ref[...], preferred_element_type=jnp.float32)
    o_ref[...] = h * _dinv(go_ref)


def _tc_mid(parts, degi, dego, b, w):
    return pl.pallas_call(
        _tc_mid_body,
        out_shape=jax.ShapeDtypeStruct((N, D), jnp.float32),
        grid=(N // BLK,),
        in_specs=[
            pl.BlockSpec((NC, BLK, D), lambda i: (0, i, 0)),
            pl.BlockSpec((BLK, 2), lambda i: (i, 0)),
            pl.BlockSpec((BLK, 2), lambda i: (i, 0)),
            pl.BlockSpec((1, D), lambda i: (0, 0)),
            pl.BlockSpec((D, D), lambda i: (0, 0)),
        ],
        out_specs=pl.BlockSpec((BLK, D), lambda i: (i, 0)),
    )(parts, degi, dego, b, w)


def _tc_final_body(p_ref, gi_ref, b_ref, o_ref):
    agg = (p_ref[0] + p_ref[1]) * _dinv(gi_ref)
    x = jnp.maximum(agg + b_ref[...], 0.0)
    m = jnp.max(x, axis=1, keepdims=True)
    e = jnp.exp(x - m)
    o_ref[...] = e / jnp.sum(e, axis=1, keepdims=True)


def _tc_final(parts, degi, b):
    return pl.pallas_call(
        _tc_final_body,
        out_shape=jax.ShapeDtypeStruct((N, D), jnp.float32),
        grid=(N // BLK,),
        in_specs=[
            pl.BlockSpec((NC, BLK, D), lambda i: (0, i, 0)),
            pl.BlockSpec((BLK, 2), lambda i: (i, 0)),
            pl.BlockSpec((1, D), lambda i: (0, 0)),
        ],
        out_specs=pl.BlockSpec((BLK, D), lambda i: (i, 0)),
    )(parts, degi, b)


def kernel(in_feat, edge_index, edge_weight, W0, b0, W1, b1, W2, b2):
    src = edge_index[0]
    dst = edge_index[1]
    e = edge_weight.shape[0]
    nb = -(-e // (NW * K))
    nb += (-nb) % 4
    epad = NW * nb * K - e

    def pad(x):
        return jnp.pad(x, (0, epad)).reshape(NW, nb, 1, K)

    sdpad = jnp.concatenate([pad(src), pad(dst)], axis=2)
    wpad = jnp.pad(edge_weight, (0, epad)).reshape(NW, nb, K)
    deg = _make_deg_kernel(nb)(sdpad, wpad)

    srca = sdpad[:, :, 0]
    dsta = sdpad[:, :, 1].reshape(NW, nb // 2, 2, K)
    wa = wpad.reshape(NW, nb // 2, 2, K)
    dego = jnp.stack([deg[0, 0], deg[1, 0]], axis=1)
    degi = jnp.stack([deg[0, 1], deg[1, 1]], axis=1)
    b0r = b0.reshape(1, D)
    b1r = b1.reshape(1, D)
    b2r = b2.reshape(1, D)

    agg_k = _make_agg_kernel(nb)
    h = _tc_first(in_feat, W0, dego)
    p = agg_k(h, srca, dsta, wa)
    h = _tc_mid(p, degi, dego, b0r, W1)
    p = agg_k(h, srca, dsta, wa)
    h = _tc_mid(p, degi, dego, b1r, W2)
    p = agg_k(h, srca, dsta, wa)
    return _tc_final(p, degi, b2r)
